# baseline (device time: 21877 ns/iter reference)
import jax
import jax.numpy as jnp
from jax import lax
from jax.experimental import pallas as pl
from jax.experimental.pallas import tpu as pltpu

N_DEV = 4


def kernel(A, B):
    M, K = A.shape
    K2, N = B.shape
    CH = M // N_DEV

    def body(a_ref, b_ref, out_ref, sbuf, rbuf, send_sems, recv_sems):
        p = lax.axis_index("i")

        barrier_sem = pltpu.get_barrier_semaphore()
        for k in range(1, N_DEV):
            pl.semaphore_signal(
                barrier_sem, inc=1,
                device_id=((p + k) % N_DEV,),
                device_id_type=pl.DeviceIdType.MESH,
            )
        pl.semaphore_wait(barrier_sem, N_DEV - 1)

        rdmas = []
        for k in (2, 1, 3):
            q = (p + k) % N_DEV
            sbuf[k - 1] = jnp.dot(
                a_ref[pl.ds(q * CH, CH), :], b_ref[...],
                preferred_element_type=jnp.float32,
            )
            rdma = pltpu.make_async_remote_copy(
                src_ref=sbuf.at[k - 1],
                dst_ref=rbuf.at[N_DEV - 1 - k],
                send_sem=send_sems.at[k - 1],
                recv_sem=recv_sems.at[N_DEV - 1 - k],
                device_id=(q,),
                device_id_type=pl.DeviceIdType.MESH,
            )
            rdma.start()
            rdmas.append(rdma)

        own = jnp.dot(
            a_ref[pl.ds(p * CH, CH), :], b_ref[...],
            preferred_element_type=jnp.float32,
        )

        for rdma in rdmas:
            rdma.wait()

        out_ref[...] = own + rbuf[0] + rbuf[1] + rbuf[2]

    return pl.pallas_call(
        body,
        out_shape=jax.ShapeDtypeStruct((CH, N), jnp.float32),
        in_specs=[
            pl.BlockSpec(memory_space=pltpu.VMEM),
            pl.BlockSpec(memory_space=pltpu.VMEM),
        ],
        out_specs=pl.BlockSpec(memory_space=pltpu.VMEM),
        scratch_shapes=[
            pltpu.VMEM((N_DEV - 1, CH, N), jnp.float32),
            pltpu.VMEM((N_DEV - 1, CH, N), jnp.float32),
            pltpu.SemaphoreType.DMA((N_DEV - 1,)),
            pltpu.SemaphoreType.DMA((N_DEV - 1,)),
        ],
        compiler_params=pltpu.CompilerParams(collective_id=0),
    )(A, B)


# device time: 15646 ns/iter; 1.3982x vs baseline; 1.3982x over previous
import jax
import jax.numpy as jnp
from jax import lax
from jax.experimental import pallas as pl
from jax.experimental.pallas import tpu as pltpu

N_DEV = 4


def kernel(A, B):
    M, K = A.shape
    K2, N = B.shape
    CH = M // N_DEV

    def body(a_ref, b_ref, out_ref, b16, sbuf, rbuf, send_sems, recv_sems):
        p = lax.axis_index("i")

        barrier_sem = pltpu.get_barrier_semaphore()
        for k in range(1, N_DEV):
            pl.semaphore_signal(
                barrier_sem, inc=1,
                device_id=((p + k) % N_DEV,),
                device_id_type=pl.DeviceIdType.MESH,
            )
        pl.semaphore_wait(barrier_sem, N_DEV - 1)

        b16[...] = b_ref[...].astype(jnp.bfloat16)

        rdmas = []
        for k in (2, 1, 3):
            q = (p + k) % N_DEV
            sbuf[k - 1] = jnp.dot(
                a_ref[pl.ds(q * CH, CH), :].astype(jnp.bfloat16), b16[...],
                preferred_element_type=jnp.float32,
            ).astype(jnp.bfloat16)
            rdma = pltpu.make_async_remote_copy(
                src_ref=sbuf.at[k - 1],
                dst_ref=rbuf.at[N_DEV - 1 - k],
                send_sem=send_sems.at[k - 1],
                recv_sem=recv_sems.at[N_DEV - 1 - k],
                device_id=(q,),
                device_id_type=pl.DeviceIdType.MESH,
            )
            rdma.start()
            rdmas.append(rdma)

        own = jnp.dot(
            a_ref[pl.ds(p * CH, CH), :].astype(jnp.bfloat16), b16[...],
            preferred_element_type=jnp.float32,
        )

        for rdma in rdmas:
            rdma.wait()

        out_ref[...] = (
            own
            + rbuf[0].astype(jnp.float32)
            + rbuf[1].astype(jnp.float32)
            + rbuf[2].astype(jnp.float32)
        )

    return pl.pallas_call(
        body,
        out_shape=jax.ShapeDtypeStruct((CH, N), jnp.float32),
        in_specs=[
            pl.BlockSpec(memory_space=pltpu.VMEM),
            pl.BlockSpec(memory_space=pltpu.VMEM),
        ],
        out_specs=pl.BlockSpec(memory_space=pltpu.VMEM),
        scratch_shapes=[
            pltpu.VMEM((K, N), jnp.bfloat16),
            pltpu.VMEM((N_DEV - 1, CH, N), jnp.bfloat16),
            pltpu.VMEM((N_DEV - 1, CH, N), jnp.bfloat16),
            pltpu.SemaphoreType.DMA((N_DEV - 1,)),
            pltpu.SemaphoreType.DMA((N_DEV - 1,)),
        ],
        compiler_params=pltpu.CompilerParams(collective_id=0),
    )(A, B)


# device time: 14858 ns/iter; 1.4724x vs baseline; 1.0530x over previous
import jax
import jax.numpy as jnp
from jax import lax
from jax.experimental import pallas as pl
from jax.experimental.pallas import tpu as pltpu

N_DEV = 4


def kernel(A, B):
    M, K = A.shape
    K2, N = B.shape
    CH = M // N_DEV

    def body(a_ref, b_ref, out_ref, b16, sbuf, rbuf, send_sems, recv_sems):
        p = lax.axis_index("i")

        barrier_sem = pltpu.get_barrier_semaphore()
        for k in range(1, N_DEV):
            pl.semaphore_signal(
                barrier_sem, inc=1,
                device_id=((p + k) % N_DEV,),
                device_id_type=pl.DeviceIdType.MESH,
            )

        b16[...] = b_ref[...].astype(jnp.bfloat16)

        def partial_chunk(q):
            return jnp.dot(
                a_ref[pl.ds(q * CH, CH), :].astype(jnp.bfloat16), b16[...],
                preferred_element_type=jnp.float32,
            )

        order = (2, 1, 3)
        rdmas = []
        for j, k in enumerate(order):
            q = (p + k) % N_DEV
            sbuf[k - 1] = partial_chunk(q).astype(jnp.bfloat16)
            if j == 0:
                pl.semaphore_wait(barrier_sem, N_DEV - 1)
            rdma = pltpu.make_async_remote_copy(
                src_ref=sbuf.at[k - 1],
                dst_ref=rbuf.at[N_DEV - 1 - k],
                send_sem=send_sems.at[k - 1],
                recv_sem=recv_sems.at[N_DEV - 1 - k],
                device_id=(q,),
                device_id_type=pl.DeviceIdType.MESH,
            )
            rdma.start()
            rdmas.append(rdma)

        acc = partial_chunk(p)
        for k, rdma in zip(order, rdmas):
            rdma.wait()
            acc = acc + rbuf[N_DEV - 1 - k].astype(jnp.float32)
        out_ref[...] = acc

    return pl.pallas_call(
        body,
        out_shape=jax.ShapeDtypeStruct((CH, N), jnp.float32),
        in_specs=[
            pl.BlockSpec(memory_space=pltpu.VMEM),
            pl.BlockSpec(memory_space=pltpu.VMEM),
        ],
        out_specs=pl.BlockSpec(memory_space=pltpu.VMEM),
        scratch_shapes=[
            pltpu.VMEM((K, N), jnp.bfloat16),
            pltpu.VMEM((N_DEV - 1, CH, N), jnp.bfloat16),
            pltpu.VMEM((N_DEV - 1, CH, N), jnp.bfloat16),
            pltpu.SemaphoreType.DMA((N_DEV - 1,)),
            pltpu.SemaphoreType.DMA((N_DEV - 1,)),
        ],
        compiler_params=pltpu.CompilerParams(collective_id=0),
    )(A, B)
